# baseline (speedup 1.0000x reference)
import dataclasses

import jax
import jax.numpy as jnp
from jax import lax
from jax.experimental import pallas as pl
from jax.experimental.pallas import tpu as pltpu
from jax.experimental.pallas import tpu_sc as plsc

_NC = 2
_NS = 16
_NW = _NC * _NS


def _constraints_tc_kernel(
    heads_ref,
    bodies_hbm,
    x_hbm,
    out_hbm,
    work_ref,
    stage_ref,
    b_smem,
    sem,
    osem,
    bsem,
):
    j = pl.program_id(0)
    cp = pltpu.make_async_copy(x_hbm.at[j], work_ref, sem)
    cp.start()
    S, H = heads_ref.shape
    K = bodies_hbm.shape[-1]
    pltpu.make_async_copy(bodies_hbm.at[0], b_smem, bsem).start()
    cp.wait()
    for s in range(S):
        pltpu.make_async_copy(bodies_hbm.at[s], b_smem, bsem).wait()

        def compute(h, _, s=s):
            m = work_ref[b_smem[h, 0]]
            for k in range(1, K):
                m = jnp.minimum(m, work_ref[b_smem[h, k]])
            stage_ref[h] = jnp.maximum(work_ref[heads_ref[s, h]], m)
            return 0

        jax.lax.fori_loop(0, H, compute, 0, unroll=16)

        if s + 1 < S:
            pltpu.make_async_copy(bodies_hbm.at[s + 1], b_smem, bsem).start()

        def apply(h, _, s=s):
            work_ref[heads_ref[s, h]] = stage_ref[h]
            return 0

        jax.lax.fori_loop(0, H, apply, 0, unroll=16)

    ocp = pltpu.make_async_copy(work_ref, out_hbm.at[j], osem)
    ocp.start()
    ocp.wait()


def _tc_part(preds_tc, heads, bodies, nsteps):
    B, C = preds_tc.shape
    S, H, K = bodies.shape
    sub = 8
    lanes = B // sub
    tbl = lanes // nsteps
    x = preds_tc.T.reshape(C, sub, nsteps, tbl).transpose(2, 0, 1, 3)
    out = pl.pallas_call(
        _constraints_tc_kernel,
        grid=(nsteps,),
        in_specs=[
            pl.BlockSpec(memory_space=pltpu.SMEM),
            pl.BlockSpec(memory_space=pl.ANY),
            pl.BlockSpec(memory_space=pl.ANY),
        ],
        out_specs=pl.BlockSpec(memory_space=pl.ANY),
        out_shape=jax.ShapeDtypeStruct((nsteps, C, sub, tbl), preds_tc.dtype),
        scratch_shapes=[
            pltpu.VMEM((C, sub, tbl), preds_tc.dtype),
            pltpu.VMEM((H, sub, tbl), preds_tc.dtype),
            pltpu.SMEM((H, K), jnp.int32),
            pltpu.SemaphoreType.DMA,
            pltpu.SemaphoreType.DMA,
            pltpu.SemaphoreType.DMA,
        ],
        compiler_params=pltpu.CompilerParams(
            dimension_semantics=("parallel",)
        ),
    )(heads, bodies, x)
    return out.transpose(1, 2, 0, 3).reshape(C, B).T


def _sc_part(preds, heads, bodies_t, b_base, b_rows):
    B, C = preds.shape
    S, K, H = bodies_t.shape
    nrows = b_rows // _NW
    mesh = plsc.VectorSubcoreMesh(core_axis_name="c", subcore_axis_name="s")

    def body(preds_hbm, heads_hbm, bodies_hbm, out_hbm,
             row0_ref, row1_ref, h_ref, b_ref, stage_ref, insem, outsem):
        rows = (row0_ref, row1_ref)
        wid = lax.axis_index("s") * _NC + lax.axis_index("c")
        base = b_base + wid * nrows
        obase = wid * nrows
        pltpu.sync_copy(heads_hbm, h_ref)
        pltpu.sync_copy(bodies_hbm, b_ref)
        pltpu.async_copy(preds_hbm.at[base], row0_ref, insem.at[0])
        pltpu.async_copy(preds_hbm.at[base + 1], row1_ref, insem.at[1])

        def do_row(r, b):
            row = base + r + b
            orow = obase + r + b
            pltpu.make_async_copy(
                preds_hbm.at[row], rows[b], insem.at[b]
            ).wait()
            for s in range(S):
                @pl.loop(0, H, step=16)
                def _(h0, s=s, b=b):
                    m = plsc.load_gather(
                        rows[b], [b_ref[s, 0, pl.ds(h0, 16)]]
                    )
                    for k in range(1, K):
                        m = jnp.minimum(
                            m,
                            plsc.load_gather(
                                rows[b], [b_ref[s, k, pl.ds(h0, 16)]]
                            ),
                        )
                    hv = plsc.load_gather(
                        rows[b], [h_ref[s, pl.ds(h0, 16)]]
                    )
                    stage_ref[pl.ds(h0, 16)] = jnp.maximum(hv, m)

                @pl.loop(0, H, step=16)
                def _(h0, s=s, b=b):
                    plsc.store_scatter(
                        rows[b],
                        [h_ref[s, pl.ds(h0, 16)]],
                        stage_ref[pl.ds(h0, 16)],
                    )

            pltpu.async_copy(rows[b], out_hbm.at[orow], outsem.at[b])

            @pl.when(r + b + 2 < nrows)
            def _():
                pltpu.make_async_copy(
                    rows[b], out_hbm.at[orow], outsem.at[b]
                ).wait()
                pltpu.async_copy(
                    preds_hbm.at[row + 2], rows[b], insem.at[b]
                )

        @pl.loop(0, nrows, step=2)
        def _(r):
            for b in range(2):
                do_row(r, b)

        for b in range(2):
            pltpu.make_async_copy(
                rows[b], out_hbm.at[obase], outsem.at[b]
            ).wait()

    cp = pltpu.CompilerParams()
    if "needs_layout_passes" in pltpu.CompilerParams.__dataclass_fields__:
        cp = dataclasses.replace(cp, needs_layout_passes=False)
    fn = pl.kernel(
        body,
        out_type=jax.ShapeDtypeStruct((b_rows, C), preds.dtype),
        mesh=mesh,
        compiler_params=cp,
        scratch_types=[
            pltpu.VMEM((C,), preds.dtype),
            pltpu.VMEM((C,), preds.dtype),
            pltpu.VMEM((S, H), jnp.int32),
            pltpu.VMEM((S, K, H), jnp.int32),
            pltpu.VMEM((H,), preds.dtype),
            pltpu.SemaphoreType.DMA((2,)),
            pltpu.SemaphoreType.DMA((2,)),
        ],
    )
    return fn(preds, heads, bodies_t)


def kernel(preds, atoms, heads, bodies):
    B, C = preds.shape
    b_sc = 1024
    b_tc = B - b_sc
    bodies_t = bodies.transpose(0, 2, 1)
    sc_out = _sc_part(preds, heads, bodies_t, b_tc, b_sc)
    tc_out = _tc_part(preds[:b_tc], heads, bodies, nsteps=3)
    return jnp.concatenate([tc_out, sc_out], axis=0)


# speedup vs baseline: 1.7800x; 1.7800x over previous
import jax
import jax.numpy as jnp
from jax.experimental import pallas as pl
from jax.experimental.pallas import tpu as pltpu


def _constraints_kernel(
    heads_ref,
    bodies_hbm,
    x_hbm,
    out_hbm,
    work_ref,
    stage_ref,
    b_smem,
    sem,
    osem,
    bsem,
):
    j = pl.program_id(0)
    cp = pltpu.make_async_copy(x_hbm.at[j], work_ref, sem)
    cp.start()
    S, H = heads_ref.shape
    K = bodies_hbm.shape[-1]
    pltpu.make_async_copy(bodies_hbm.at[0], b_smem, bsem).start()
    cp.wait()
    for s in range(S):
        pltpu.make_async_copy(bodies_hbm.at[s], b_smem, bsem).wait()

        def compute(h, _, s=s):
            m = work_ref[b_smem[h, 0]]
            for k in range(1, K):
                m = jnp.minimum(m, work_ref[b_smem[h, k]])
            stage_ref[h] = jnp.maximum(work_ref[heads_ref[s, h]], m)
            return 0

        jax.lax.fori_loop(0, H, compute, 0, unroll=16)

        if s + 1 < S:
            pltpu.make_async_copy(bodies_hbm.at[s + 1], b_smem, bsem).start()

        def apply(h, _, s=s):
            work_ref[heads_ref[s, h]] = stage_ref[h]
            return 0

        jax.lax.fori_loop(0, H, apply, 0, unroll=16)

    ocp = pltpu.make_async_copy(work_ref, out_hbm.at[j], osem)
    ocp.start()
    ocp.wait()


def kernel(preds, atoms, heads, bodies):
    B, C = preds.shape
    S, H, K = bodies.shape
    sub = 8
    lanes = B // sub
    tbl = min(256, lanes)
    nsteps = lanes // tbl
    x = (
        preds.T.reshape(C, sub, nsteps, tbl)
        .transpose(2, 0, 1, 3)
    )
    grid = (nsteps,)
    out = pl.pallas_call(
        _constraints_kernel,
        grid=grid,
        in_specs=[
            pl.BlockSpec(memory_space=pltpu.SMEM),
            pl.BlockSpec(memory_space=pl.ANY),
            pl.BlockSpec(memory_space=pl.ANY),
        ],
        out_specs=pl.BlockSpec(memory_space=pl.ANY),
        out_shape=jax.ShapeDtypeStruct((nsteps, C, sub, tbl), preds.dtype),
        scratch_shapes=[
            pltpu.VMEM((C, sub, tbl), preds.dtype),
            pltpu.VMEM((H, sub, tbl), preds.dtype),
            pltpu.SMEM((H, K), jnp.int32),
            pltpu.SemaphoreType.DMA,
            pltpu.SemaphoreType.DMA,
            pltpu.SemaphoreType.DMA,
        ],
        compiler_params=pltpu.CompilerParams(
            dimension_semantics=("parallel",)
        ),
    )(heads, bodies, x)
    return out.transpose(1, 2, 0, 3).reshape(C, B).T
